# Initial kernel scaffold; baseline (speedup 1.0000x reference)
import jax
import jax.numpy as jnp
from jax.experimental import pallas as pl


def kernel(hidden_states, attention_mask, ln1_w, ln2_w, wq, wk, wv, wo, wr, w1, w2, w3, position_ids):
    raise NotImplementedError("write your pallas kernel here")


# speedup vs baseline: 1.4900x; 1.4900x over previous
import functools

import jax
import jax.numpy as jnp
import numpy as np
from jax.experimental import pallas as pl
from jax.experimental.pallas import tpu as pltpu

B, S, D, H, KV, DH, E, TOPK, F = 1, 2048, 2048, 16, 8, 128, 8, 2, 2048
T = S
EPS = 1e-5
THETA = 1000000.0
HALF = DH // 2
BS = 256
NBS = S // BS
BQ = 256
NQ = S // BQ
BLK = 256
A = T * TOPK
NB = (A + E * (BLK - 1) + BLK - 1) // BLK
P = NB * BLK
NF = 2
FT = F // NF


def _k1_body(hid_ref, w_ref, wq_ref, wk_ref, wv_ref, q_ref, k_ref, v_ref):
    sb = pl.program_id(0)
    x = hid_ref[...]
    var = jnp.mean(x * x, axis=1, keepdims=True)
    h = (x * jax.lax.rsqrt(var + EPS) * w_ref[...]).astype(jnp.bfloat16)
    q = jnp.dot(h, wq_ref[...], preferred_element_type=jnp.float32)
    k = jnp.dot(h, wk_ref[...], preferred_element_type=jnp.float32)
    v = jnp.dot(h, wv_ref[...], preferred_element_type=jnp.float32)
    pos = (sb * BS + jax.lax.broadcasted_iota(jnp.int32, (BS, 1, 1), 0)).astype(jnp.float32)
    inv = jnp.exp(jax.lax.broadcasted_iota(jnp.int32, (1, 1, HALF), 2).astype(jnp.float32)
                  * (np.float32(-np.log(THETA) / HALF)))
    ang = pos * inv
    cos = jnp.cos(ang)
    sin = jnp.sin(ang)

    def rope(xh, nh):
        xh = xh.reshape(BS, nh, DH)
        x1 = xh[..., :HALF]
        x2 = xh[..., HALF:]
        return jnp.concatenate([x1 * cos - x2 * sin, x2 * cos + x1 * sin], axis=-1)

    q_ref[...] = rope(q, H).reshape(BS, H * DH).astype(jnp.bfloat16)
    k_ref[...] = rope(k, KV).transpose(1, 0, 2).astype(jnp.bfloat16)
    v_ref[...] = v.reshape(BS, KV, DH).transpose(1, 0, 2).astype(jnp.bfloat16)


def _k2_body(q_ref, k_ref, v_ref, o_ref):
    qb = pl.program_id(1)
    q = q_ref[...]
    k = k_ref[0]
    v = v_ref[0]
    s = jax.lax.dot_general(q, k, (((1,), (1,)), ((), ())),
                            preferred_element_type=jnp.float32)
    s = s * (1.0 / np.sqrt(DH).astype(np.float32))
    rows = qb * BQ + jax.lax.broadcasted_iota(jnp.int32, (BQ, S), 0)
    cols = jax.lax.broadcasted_iota(jnp.int32, (BQ, S), 1)
    s = jnp.where(cols <= rows, s, -1e9)
    m = jnp.max(s, axis=1, keepdims=True)
    p = jnp.exp(s - m)
    l = jnp.sum(p, axis=1, keepdims=True)
    a = (p / l).astype(jnp.bfloat16)
    o_ref[...] = jnp.dot(a, v, preferred_element_type=jnp.float32).astype(jnp.bfloat16)


def _k3_body(o_ref, res_ref, w_ref, wo_ref, wr_ref, x_ref, h2_ref, meta_ref):
    o = o_ref[...]
    x = res_ref[...] + jnp.dot(o, wo_ref[...], preferred_element_type=jnp.float32)
    x_ref[...] = x
    var = jnp.mean(x * x, axis=1, keepdims=True)
    h2 = x * jax.lax.rsqrt(var + EPS) * w_ref[...]
    h2_ref[...] = h2.astype(jnp.bfloat16)
    l = jnp.dot(h2.astype(jnp.bfloat16), wr_ref[...].astype(jnp.bfloat16),
                preferred_element_type=jnp.float32)
    col = jax.lax.broadcasted_iota(jnp.int32, (BS, 128), 1)
    l = jnp.where(col < E, l, -1e9)
    m1 = jnp.max(l, axis=1, keepdims=True)
    i1 = jnp.min(jnp.where(l >= m1, col, 999), axis=1, keepdims=True)
    lm = jnp.where(col == i1, -1e9, l)
    m2 = jnp.max(lm, axis=1, keepdims=True)
    i2 = jnp.min(jnp.where(lm >= m2, col, 999), axis=1, keepdims=True)
    e2 = jnp.exp(m2 - m1)
    wa = 1.0 / (1.0 + e2)
    wb = e2 / (1.0 + e2)
    meta_ref[...] = (jnp.where(col == 0, i1.astype(jnp.float32), 0.0)
                     + jnp.where(col == 1, i2.astype(jnp.float32), 0.0)
                     + jnp.where(col == 2, wa, 0.0)
                     + jnp.where(col == 3, wb, 0.0))


def _k3b_body(meta_ref, disp_ref, eb_ref):
    i1 = meta_ref[:, 0:1]
    i2 = meta_ref[:, 1:2]
    ecol = jax.lax.broadcasted_iota(jnp.int32, (1, E), 1).astype(jnp.float32)
    oh1 = (i1 == ecol).astype(jnp.float32)
    oh2 = (i2 == ecol).astype(jnp.float32)
    ohtok = (oh1 + oh2).astype(jnp.bfloat16)
    r2 = jax.lax.broadcasted_iota(jnp.int32, (256, 256), 0)
    c2 = jax.lax.broadcasted_iota(jnp.int32, (256, 256), 1)
    ltri = (c2 < r2).astype(jnp.bfloat16)
    ones = jnp.ones((1, 256), jnp.bfloat16)
    carry = jnp.zeros((1, E), jnp.float32)
    parts = []
    for b in range(S // 256):
        blk = ohtok[b * 256:(b + 1) * 256]
        cs = jnp.dot(ltri, blk, preferred_element_type=jnp.float32) + carry
        parts.append(cs)
        carry = carry + jnp.dot(ones, blk, preferred_element_type=jnp.float32)
    csum = jnp.concatenate(parts, axis=0)
    counts = carry.astype(jnp.int32)
    blocks_e = (counts + (BLK - 1)) // BLK
    j8 = jax.lax.broadcasted_iota(jnp.int32, (E, E), 0)
    e8 = jax.lax.broadcasted_iota(jnp.int32, (E, E), 1)
    bsrc = jnp.broadcast_to(blocks_e.reshape(E, 1), (E, E))
    blk_start = jnp.sum(jnp.where(j8 < e8, bsrc, 0), axis=0, keepdims=True)
    offf = (blk_start * BLK).astype(jnp.float32)
    pos0 = jnp.sum(oh1 * (offf + csum), axis=1, keepdims=True)
    pos1 = jnp.sum(oh2 * (offf + csum), axis=1, keepdims=True)
    col = jax.lax.broadcasted_iota(jnp.int32, (S, 128), 1)
    disp_ref[...] = (jnp.where(col == 0, pos0, 0.0)
                     + jnp.where(col == 1, pos1, 0.0)).astype(jnp.int32)
    bi = jax.lax.broadcasted_iota(jnp.int32, (32, E), 0)
    ebv = jnp.sum((jnp.broadcast_to(blk_start, (32, E)) <= bi).astype(jnp.int32),
                  axis=1, keepdims=True) - 1
    eb_ref[...] = jnp.broadcast_to(ebv, (32, 128))


def _k4_body(eb_ref, x_ref, w1_ref, w3_ref, gs_ref):
    x = x_ref[...]
    w1 = w1_ref[0].astype(jnp.bfloat16)
    w3 = w3_ref[0].astype(jnp.bfloat16)
    a = jnp.dot(x, w1, preferred_element_type=jnp.float32)
    c = jnp.dot(x, w3, preferred_element_type=jnp.float32)
    g = (a * (1.0 / (1.0 + jnp.exp(-a)))) * c
    gs_ref[...] = g.astype(jnp.bfloat16)


def _k5_body(eb_ref, g_ref, w2_ref, ys_ref):
    w2 = w2_ref[0].astype(jnp.bfloat16)
    y = jnp.dot(g_ref[...], w2, preferred_element_type=jnp.float32)
    ys_ref[...] = y.astype(jnp.bfloat16)


def _k6_body(x_ref, meta_ref, a_ref, b_ref, o_ref):
    wa = meta_ref[:, 2:3]
    wb = meta_ref[:, 3:4]
    o_ref[...] = (x_ref[...] + wa * a_ref[...].astype(jnp.float32)
                  + wb * b_ref[...].astype(jnp.float32))


def kernel(hidden_states, attention_mask, ln1_w, ln2_w, wq, wk, wv, wo, wr, w1, w2, w3, position_ids):
    del attention_mask, position_ids
    hid = hidden_states.reshape(S, D)
    ln1 = ln1_w.reshape(1, D)
    ln2 = ln2_w.reshape(1, D)
    wqb = wq.astype(jnp.bfloat16)
    wkb = wk.astype(jnp.bfloat16)
    wvb = wv.astype(jnp.bfloat16)
    wob = wo.astype(jnp.bfloat16)
    wr_pad = jnp.zeros((D, 128), jnp.float32).at[:, :E].set(wr)

    q, k, v = pl.pallas_call(
        _k1_body,
        grid=(NBS,),
        in_specs=[
            pl.BlockSpec((BS, D), lambda i: (i, 0)),
            pl.BlockSpec((1, D), lambda i: (0, 0)),
            pl.BlockSpec((D, H * DH), lambda i: (0, 0)),
            pl.BlockSpec((D, KV * DH), lambda i: (0, 0)),
            pl.BlockSpec((D, KV * DH), lambda i: (0, 0)),
        ],
        out_specs=[
            pl.BlockSpec((BS, H * DH), lambda i: (i, 0)),
            pl.BlockSpec((KV, BS, DH), lambda i: (0, i, 0)),
            pl.BlockSpec((KV, BS, DH), lambda i: (0, i, 0)),
        ],
        out_shape=[
            jax.ShapeDtypeStruct((S, H * DH), jnp.bfloat16),
            jax.ShapeDtypeStruct((KV, S, DH), jnp.bfloat16),
            jax.ShapeDtypeStruct((KV, S, DH), jnp.bfloat16),
        ],
    )(hid, ln1, wqb, wkb, wvb)

    o = pl.pallas_call(
        _k2_body,
        grid=(H, NQ),
        in_specs=[
            pl.BlockSpec((BQ, DH), lambda h, qb: (qb, h)),
            pl.BlockSpec((1, S, DH), lambda h, qb: (h // 2, 0, 0)),
            pl.BlockSpec((1, S, DH), lambda h, qb: (h // 2, 0, 0)),
        ],
        out_specs=pl.BlockSpec((BQ, DH), lambda h, qb: (qb, h)),
        out_shape=jax.ShapeDtypeStruct((S, H * DH), jnp.bfloat16),
    )(q, k, v)

    x, h2b, meta = pl.pallas_call(
        _k3_body,
        grid=(NBS,),
        in_specs=[
            pl.BlockSpec((BS, H * DH), lambda i: (i, 0)),
            pl.BlockSpec((BS, D), lambda i: (i, 0)),
            pl.BlockSpec((1, D), lambda i: (0, 0)),
            pl.BlockSpec((H * DH, D), lambda i: (0, 0)),
            pl.BlockSpec((D, 128), lambda i: (0, 0)),
        ],
        out_specs=[
            pl.BlockSpec((BS, D), lambda i: (i, 0)),
            pl.BlockSpec((BS, D), lambda i: (i, 0)),
            pl.BlockSpec((BS, 128), lambda i: (i, 0)),
        ],
        out_shape=[
            jax.ShapeDtypeStruct((S, D), jnp.float32),
            jax.ShapeDtypeStruct((S, D), jnp.bfloat16),
            jax.ShapeDtypeStruct((S, 128), jnp.float32),
        ],
    )(o, hid, ln2, wob, wr_pad)

    disp, eb32 = pl.pallas_call(
        _k3b_body,
        grid=(1,),
        in_specs=[pl.BlockSpec((S, 128), lambda i: (0, 0))],
        out_specs=[
            pl.BlockSpec((S, 128), lambda i: (0, 0)),
            pl.BlockSpec((32, 128), lambda i: (0, 0)),
        ],
        out_shape=[
            jax.ShapeDtypeStruct((S, 128), jnp.int32),
            jax.ShapeDtypeStruct((32, 128), jnp.int32),
        ],
    )(meta)

    eb = eb32[:NB, 0]
    pos = disp[:, 0:2].reshape(A)

    tok = jnp.arange(A, dtype=jnp.int32) // TOPK
    xs = jnp.zeros((P, D), jnp.bfloat16).at[pos].set(jnp.take(h2b, tok, axis=0))

    gs = pl.pallas_call(
        _k4_body,
        grid_spec=pltpu.PrefetchScalarGridSpec(
            num_scalar_prefetch=1,
            grid=(NF, NB),
            in_specs=[
                pl.BlockSpec((BLK, D), lambda f, b, eb_r: (b, 0)),
                pl.BlockSpec((1, D, FT), lambda f, b, eb_r: (eb_r[b], 0, f)),
                pl.BlockSpec((1, D, FT), lambda f, b, eb_r: (eb_r[b], 0, f)),
            ],
            out_specs=pl.BlockSpec((BLK, FT), lambda f, b, eb_r: (b, f)),
        ),
        out_shape=jax.ShapeDtypeStruct((P, F), jnp.bfloat16),
    )(eb, xs, w1, w3)

    ys = pl.pallas_call(
        _k5_body,
        grid_spec=pltpu.PrefetchScalarGridSpec(
            num_scalar_prefetch=1,
            grid=(NB,),
            in_specs=[
                pl.BlockSpec((BLK, F), lambda b, eb_r: (b, 0)),
                pl.BlockSpec((1, F, D), lambda b, eb_r: (eb_r[b], 0, 0)),
            ],
            out_specs=pl.BlockSpec((BLK, D), lambda b, eb_r: (b, 0)),
        ),
        out_shape=jax.ShapeDtypeStruct((P, D), jnp.bfloat16),
    )(eb, gs, w2)

    yg = jnp.take(ys, pos, axis=0).reshape(T, TOPK, D)
    yg0 = yg[:, 0]
    yg1 = yg[:, 1]

    out = pl.pallas_call(
        _k6_body,
        grid=(NBS,),
        in_specs=[
            pl.BlockSpec((BS, D), lambda i: (i, 0)),
            pl.BlockSpec((BS, 128), lambda i: (i, 0)),
            pl.BlockSpec((BS, D), lambda i: (i, 0)),
            pl.BlockSpec((BS, D), lambda i: (i, 0)),
        ],
        out_specs=pl.BlockSpec((BS, D), lambda i: (i, 0)),
        out_shape=jax.ShapeDtypeStruct((S, D), jnp.float32),
    )(x, meta, yg0, yg1)
    return out.reshape(B, S, D)
